# Initial kernel scaffold; baseline (speedup 1.0000x reference)
import jax
import jax.numpy as jnp
from jax.experimental import pallas as pl


def kernel(x, rays_o, xy_plane, yz_plane, xz_plane):
    raise NotImplementedError("write your pallas kernel here")


# speedup vs baseline: 2.8600x; 2.8600x over previous
import functools

import jax
import jax.numpy as jnp
from jax import lax
from jax.experimental import pallas as pl
from jax.experimental.pallas import tpu as pltpu
from jax.experimental.pallas import tpu_sc as plsc

FEAT = 64
SIZE = 512
LANES = 16
CHUNK = 64
NBUF = 2


def _pixel(c):
    p = (c + 1.0) * 0.5 * (SIZE - 1)
    i0 = p.astype(jnp.int32)
    f = p - i0.astype(jnp.float32)
    i0 = jnp.minimum(jnp.maximum(i0, 0), SIZE - 1)
    i1 = jnp.minimum(i0 + 1, SIZE - 1)
    return i0, i1, f


def _sc_embed(xt, txy, txz, tyz, B, nw):
    pts_per_w = B // nw
    nch = pts_per_w // CHUNK
    mesh = plsc.VectorSubcoreMesh(core_axis_name="c", subcore_axis_name="s")

    @functools.partial(
        pl.kernel,
        out_type=jax.ShapeDtypeStruct((B, FEAT), jnp.float32),
        mesh=mesh,
        compiler_params=pltpu.CompilerParams(
            needs_layout_passes=False, use_tc_tiling_on_sc=False),
        scratch_types=[
            pltpu.VMEM((NBUF, 3, CHUNK), jnp.float32),
            pltpu.VMEM((NBUF, 12, CHUNK), jnp.int32),
            pltpu.VMEM((NBUF, LANES, CHUNK), jnp.float32),
            pltpu.VMEM((NBUF, 12, CHUNK, FEAT), jnp.float32),
            pltpu.VMEM((NBUF, CHUNK, FEAT), jnp.float32),
            pltpu.SemaphoreType.DMA,
            pltpu.SemaphoreType.DMA,
            pltpu.SemaphoreType.DMA,
        ],
    )
    def k(xt_h, txy_h, txz_h, tyz_h, out_h, coords, idx, wgt, rows, obuf,
          sem0, sem1, semc):
        wid = lax.axis_index("s") * 2 + lax.axis_index("c")
        w_base = wid * pts_per_w
        tabs = (txy_h, txz_h, tyz_h)
        sems = (sem0, sem1)

        def compute_idx(kk, b):
            base = w_base + kk * CHUNK
            for d in range(3):
                pltpu.async_copy(xt_h.at[d, pl.ds(base, CHUNK)],
                                 coords.at[b, d], semc)
            for d in range(3):
                pltpu.make_async_copy(xt_h.at[d, pl.ds(base, CHUNK)],
                                      coords.at[b, d], semc).wait()
            for j in range(CHUNK // LANES):
                sl = pl.ds(j * LANES, LANES)
                iw = [None] * 3
                ih = [None] * 3
                fr = [None] * 3
                for d in range(3):
                    iw[d], ih[d], fr[d] = _pixel(coords[b, d, sl])
                for p, (dw, dh) in enumerate(((0, 1), (0, 2), (1, 2))):
                    w0, w1, fw = iw[dw], ih[dw], fr[dw]
                    h0, h1, fh = iw[dh], ih[dh], fr[dh]
                    r00 = h0 * SIZE + w0
                    r01 = h0 * SIZE + w1
                    r10 = h1 * SIZE + w0
                    r11 = h1 * SIZE + w1
                    gw = 1.0 - fw
                    gh = 1.0 - fh
                    idx[b, 4 * p + 0, sl] = r00
                    idx[b, 4 * p + 1, sl] = r01
                    idx[b, 4 * p + 2, sl] = r10
                    idx[b, 4 * p + 3, sl] = r11
                    wgt[b, 4 * p + 0, sl] = gh * gw
                    wgt[b, 4 * p + 1, sl] = gh * fw
                    wgt[b, 4 * p + 2, sl] = fh * gw
                    wgt[b, 4 * p + 3, sl] = fh * fw

        def start_gathers(b):
            for p in range(3):
                for c in range(4):
                    s = 4 * p + c
                    pltpu.async_copy(tabs[p].at[idx.at[b, s]],
                                     rows.at[b, s], sems[b])

        def wait_gathers(b):
            for p in range(3):
                for c in range(4):
                    s = 4 * p + c
                    pltpu.make_async_copy(tabs[p].at[idx.at[b, s]],
                                          rows.at[b, s], sems[b]).wait()

        def combine(b):
            iota16 = lax.iota(jnp.int32, LANES)

            def body(g, _):
                wv = plsc.load_gather(
                    wgt.at[b], [iota16, jnp.full((LANES,), 0, jnp.int32) + g])
                for q in range(FEAT // LANES):
                    sl = pl.ds(q * LANES, LANES)
                    acc = wv[0] * rows[b, 0, g, sl]
                    for s in range(1, 12):
                        acc = acc + wv[s] * rows[b, s, g, sl]
                    obuf[b, g, sl] = acc
                return 0
            lax.fori_loop(0, CHUNK, body, 0)

        compute_idx(0, 0)
        start_gathers(0)

        def outer(i, _):
            for b in range(NBUF):
                kk = i * NBUF + b
                nb = (b + 1) % NBUF

                @pl.when(kk + 1 < nch)
                def _():
                    compute_idx(kk + 1, nb)
                    start_gathers(nb)

                wait_gathers(b)
                combine(b)
                pltpu.sync_copy(obuf.at[b],
                                out_h.at[pl.ds(w_base + kk * CHUNK, CHUNK)])
            return 0

        lax.fori_loop(0, nch // NBUF, outer, 0)

    return k(xt, txy, txz, tyz)


def kernel(x, rays_o, xy_plane, yz_plane, xz_plane):
    del rays_o
    B = x.shape[0]
    info = plsc.get_sparse_core_info()
    nw = info.num_cores * info.num_subcores
    xt = x.T

    def tab(plane):
        return plane[0].transpose(1, 2, 0).reshape(SIZE * SIZE, FEAT)

    return _sc_embed(xt, tab(xy_plane), tab(xz_plane), tab(yz_plane), B, nw)
